# Initial kernel scaffold; baseline (speedup 1.0000x reference)
import jax
import jax.numpy as jnp
from jax.experimental import pallas as pl


def kernel(x, edge_index, edge_type):
    raise NotImplementedError("write your pallas kernel here")


# speedup vs baseline: 5.8760x; 5.8760x over previous
import jax
import jax.numpy as jnp
from jax import lax
from jax.experimental import pallas as pl
from jax.experimental.pallas import tpu as pltpu
from jax.experimental.pallas import tpu_sc as plsc

_R = 3
_N = 10000
_NP = 10240
_E = 320000
_EP = 327680
_EROWS = _EP // 128
_TROWS = _EROWS // 16
_MR = 16
_MACROS = _TROWS // _MR
_ACC = _R * _NP
_STRIPE = _ACC // 16
_QW = 32


def _sc_body(xf, cols, offs, sums, hists,
             offb, colb, gixb, rows0, rows1, hist,
             acc, dsem0, dsem1, gsem, ssem):
    c = lax.axis_index("c")
    s = lax.axis_index("s")
    zeros16 = jnp.zeros((16,), jnp.float32)
    ones16 = jnp.ones((16,), jnp.float32)

    def zh(r, carry):
        hist[pl.ds(r * 16, 16)] = zeros16
        return carry
    lax.fori_loop(0, 2048, zh, 0)

    base_a = s * _STRIPE

    for p in range(2):
        q = p * 2 + c
        qbase = q * _N

        def z0(r, carry):
            for w in range(2):
                rows0[r, pl.ds(w * 16, 16)] = zeros16
            return carry
        lax.fori_loop(0, 128, z0, 0)

        def za(k, carry):
            pltpu.sync_copy(rows0, acc.at[pl.ds(base_a + k * 128, 128)])
            return carry
        lax.fori_loop(0, _STRIPE // 128, za, 0)

        plsc.subcore_barrier()

        def macro(m, carry):
            base = s * _TROWS + m * _MR
            cp0 = pltpu.async_copy(offs.at[pl.ds(base, _MR)], offb, dsem0)
            cp1 = pltpu.async_copy(cols.at[pl.ds(base, _MR)], colb, dsem1)
            cp0.wait()
            cp1.wait()
            for j in range(_MR):
                for w in range(8):
                    gixb[j, pl.ds(w * 16, 16)] = (
                        colb[j, pl.ds(w * 16, 16)] + qbase)
                rb = rows0 if j % 2 == 0 else rows1
                pltpu.async_copy(xf.at[gixb.at[j]], rb, gsem).wait()
                pltpu.async_copy(rb, acc.at[offb.at[j]], ssem,
                                 add=True).wait()
                if p == 0:
                    for w in range(8):
                        o = offb[j, pl.ds(w * 16, 16)]
                        plsc.addupdate_scatter(hist, [o], ones16)
            return carry
        lax.fori_loop(0, _MACROS, macro, 0)

        plsc.subcore_barrier()

        pltpu.sync_copy(acc.at[pl.ds(base_a, _STRIPE)],
                        sums.at[pl.ds(q * _ACC + base_a, _STRIPE)])

    @pl.when(c == 0)
    def _():
        pltpu.sync_copy(hist, hists.at[s])


_sc_call = pl.kernel(
    _sc_body,
    out_type=[
        jax.ShapeDtypeStruct((4 * _ACC, _QW), jnp.float32),
        jax.ShapeDtypeStruct((16, 32768), jnp.float32),
    ],
    mesh=plsc.VectorSubcoreMesh(core_axis_name="c", subcore_axis_name="s"),
    compiler_params=pltpu.CompilerParams(
        needs_layout_passes=False, use_tc_tiling_on_sc=False),
    scratch_types=[
        pltpu.VMEM((_MR, 128), jnp.int32),
        pltpu.VMEM((_MR, 128), jnp.int32),
        pltpu.VMEM((_MR, 128), jnp.int32),
        pltpu.VMEM((128, _QW), jnp.float32),
        pltpu.VMEM((128, _QW), jnp.float32),
        pltpu.VMEM((32768,), jnp.float32),
        pltpu.VMEM_SHARED((_ACC, _QW), jnp.float32),
        pltpu.SemaphoreType.DMA,
        pltpu.SemaphoreType.DMA,
        pltpu.SemaphoreType.DMA,
        pltpu.SemaphoreType.DMA,
    ],
)


def _fin_body(x_ref, s_ref, cnt_ref, o_ref):
    cnt = jnp.sum(cnt_ref[...].reshape(16, _R, _FB), axis=0)
    x = x_ref[...]
    c0 = jnp.maximum(cnt[0], 1.0)[:, None]
    c1 = jnp.maximum(cnt[1], 1.0)[:, None]
    c2 = jnp.maximum(cnt[2], 1.0)[:, None]
    ctx = s_ref[0] / c0 + s_ref[1] / c1 + s_ref[2] / c2
    rc = (jnp.minimum(cnt[0], 1.0) + jnp.minimum(cnt[1], 1.0)
          + jnp.minimum(cnt[2], 1.0))
    rc = jnp.maximum(rc, 1.0)[:, None]
    o_ref[...] = x - ctx / rc


_FB = 1024

_fin_call = pl.pallas_call(
    _fin_body,
    grid=(_NP // _FB,),
    in_specs=[
        pl.BlockSpec((_FB, 128), lambda i: (i, 0)),
        pl.BlockSpec((_R, _FB, 128), lambda i: (0, i, 0)),
        pl.BlockSpec((48, _FB), lambda i: (0, i)),
    ],
    out_specs=pl.BlockSpec((_FB, 128), lambda i: (i, 0)),
    out_shape=jax.ShapeDtypeStruct((_NP, 128), jnp.float32),
)


def kernel(x, edge_index, edge_type):
    row = edge_index[0]
    col = edge_index[1]
    off = edge_type * _NP + row
    pad = _EP - _E
    dummy_off = _N + (jnp.arange(pad, dtype=jnp.int32) % 16)
    offp = jnp.concatenate([off, dummy_off]).reshape(_EROWS, 128)
    colp = jnp.concatenate(
        [col, jnp.zeros((pad,), jnp.int32)]).reshape(_EROWS, 128)
    xf = x.reshape(_N, 4, _QW).transpose(1, 0, 2).reshape(4 * _N, _QW)

    sums, hists = _sc_call(xf, colp, offp)
    s = (sums.reshape(4, _R, _NP, _QW)
         .transpose(1, 2, 0, 3).reshape(_R, _NP, 128))
    cnt = hists[:, :_ACC].reshape(16 * _R, _NP)
    xpad = jnp.concatenate(
        [x, jnp.zeros((_NP - _N, 128), jnp.float32)], axis=0)
    out = _fin_call(xpad, s, cnt)
    return out[:_N]


# speedup vs baseline: 8.2495x; 1.4039x over previous
import jax
import jax.numpy as jnp
from jax import lax
from jax.experimental import pallas as pl
from jax.experimental.pallas import tpu as pltpu
from jax.experimental.pallas import tpu_sc as plsc

_R = 3
_N = 10000
_NP = 10240
_E = 320000
_EP = 327680
_EROWS = _EP // 128
_TROWS = _EROWS // 16
_MR = 16
_MACROS = _TROWS // _MR
_ACC = _R * _NP
_STRIPE = _ACC // 16
_QW = 32


_NBUF = 4


def _sc_body(xf, gidx, offs, sums, hists,
             offb, gixb, rows, hist,
             dsem0, dsem1, gsems, ssems, acc):
    c = lax.axis_index("c")
    s = lax.axis_index("s")
    zeros16 = jnp.zeros((16,), jnp.float32)
    ones16 = jnp.ones((16,), jnp.float32)

    def zh(r, carry):
        hist[pl.ds(r * 16, 16)] = zeros16
        return carry
    lax.fori_loop(0, 2048, zh, 0)

    base_a = s * _STRIPE

    for p in range(2):
        q = p * 2 + c

        def z0(r, carry):
            for w in range(2):
                rows[r, pl.ds(w * 16, 16)] = zeros16
            return carry
        lax.fori_loop(0, 128, z0, 0)

        def za(k, carry):
            pltpu.sync_copy(rows.at[pl.ds(0, 128)],
                            acc.at[pl.ds(base_a + k * 128, 128)])
            return carry
        lax.fori_loop(0, _STRIPE // 128, za, 0)

        plsc.subcore_barrier()

        def macro(m, carry):
            base = s * _TROWS + m * _MR
            cp0 = pltpu.async_copy(offs.at[pl.ds(base, _MR)], offb, dsem0)
            cp1 = pltpu.async_copy(
                gidx.at[pl.ds(q * _EROWS + base, _MR)], gixb, dsem1)
            cp0.wait()
            cp1.wait()
            gd, sd = {}, {}
            for b in range(_NBUF):
                gd[b] = pltpu.async_copy(
                    xf.at[gixb.at[b]], rows.at[pl.ds(b * 128, 128)],
                    gsems.at[b])
            for j in range(_MR):
                b = j % _NBUF
                gd[j].wait()
                sd[j] = pltpu.async_copy(
                    rows.at[pl.ds(b * 128, 128)], acc.at[offb.at[j]],
                    ssems.at[b], add=True)
                if p == 0:
                    for w in range(8):
                        o = offb[j, pl.ds(w * 16, 16)]
                        plsc.addupdate_scatter(hist, [o], ones16)
                nxt = j + _NBUF
                if nxt < _MR:
                    sd[j].wait()
                    gd[nxt] = pltpu.async_copy(
                        xf.at[gixb.at[nxt]], rows.at[pl.ds(b * 128, 128)],
                        gsems.at[b])
            for j in range(_MR - _NBUF, _MR):
                sd[j].wait()
            return carry
        lax.fori_loop(0, _MACROS, macro, 0)

        plsc.subcore_barrier()

        pltpu.sync_copy(acc.at[pl.ds(base_a, _STRIPE)],
                        sums.at[pl.ds(q * _ACC + base_a, _STRIPE)])

    @pl.when(c == 0)
    def _():
        pltpu.sync_copy(hist, hists.at[s])


_sc_call = pl.kernel(
    _sc_body,
    out_type=[
        jax.ShapeDtypeStruct((4 * _ACC, _QW), jnp.float32),
        jax.ShapeDtypeStruct((16, 32768), jnp.float32),
    ],
    mesh=plsc.VectorSubcoreMesh(core_axis_name="c", subcore_axis_name="s"),
    compiler_params=pltpu.CompilerParams(
        needs_layout_passes=False, use_tc_tiling_on_sc=False),
    scratch_types=[
        pltpu.VMEM((_MR, 128), jnp.int32),
        pltpu.VMEM((_MR, 128), jnp.int32),
        pltpu.VMEM((_NBUF * 128, _QW), jnp.float32),
        pltpu.VMEM((32768,), jnp.float32),
        pltpu.SemaphoreType.DMA,
        pltpu.SemaphoreType.DMA,
        pltpu.SemaphoreType.DMA((_NBUF,)),
        pltpu.SemaphoreType.DMA((_NBUF,)),
        pltpu.VMEM_SHARED((_ACC, _QW), jnp.float32),
    ],
)


def _fin_body(x_ref, s_ref, cnt_ref, o_ref):
    cnt = jnp.sum(cnt_ref[...].reshape(16, _R, _FB), axis=0)
    x = x_ref[...]
    c0 = jnp.maximum(cnt[0], 1.0)[:, None]
    c1 = jnp.maximum(cnt[1], 1.0)[:, None]
    c2 = jnp.maximum(cnt[2], 1.0)[:, None]
    ctx = s_ref[0] / c0 + s_ref[1] / c1 + s_ref[2] / c2
    rc = (jnp.minimum(cnt[0], 1.0) + jnp.minimum(cnt[1], 1.0)
          + jnp.minimum(cnt[2], 1.0))
    rc = jnp.maximum(rc, 1.0)[:, None]
    o_ref[...] = x - ctx / rc


_FB = 1024

_fin_call = pl.pallas_call(
    _fin_body,
    grid=(_NP // _FB,),
    in_specs=[
        pl.BlockSpec((_FB, 128), lambda i: (i, 0)),
        pl.BlockSpec((_R, _FB, 128), lambda i: (0, i, 0)),
        pl.BlockSpec((48, _FB), lambda i: (0, i)),
    ],
    out_specs=pl.BlockSpec((_FB, 128), lambda i: (i, 0)),
    out_shape=jax.ShapeDtypeStruct((_NP, 128), jnp.float32),
)


def kernel(x, edge_index, edge_type):
    row = edge_index[0]
    col = edge_index[1]
    off = edge_type * _NP + row
    pad = _EP - _E
    dummy_off = _N + (jnp.arange(pad, dtype=jnp.int32) % 16)
    offp = jnp.concatenate([off, dummy_off]).reshape(_EROWS, 128)
    colp = jnp.concatenate([col, jnp.zeros((pad,), jnp.int32)])
    gidx = (colp[None, :]
            + (jnp.arange(4, dtype=jnp.int32) * _N)[:, None]
            ).reshape(4 * _EROWS, 128)
    xf = x.reshape(_N, 4, _QW).transpose(1, 0, 2).reshape(4 * _N, _QW)

    sums, hists = _sc_call(xf, gidx, offp)
    s = (sums.reshape(4, _R, _NP, _QW)
         .transpose(1, 2, 0, 3).reshape(_R, _NP, 128))
    cnt = hists[:, :_ACC].reshape(16 * _R, _NP)
    xpad = jnp.concatenate(
        [x, jnp.zeros((_NP - _N, 128), jnp.float32)], axis=0)
    out = _fin_call(xpad, s, cnt)
    return out[:_N]


# speedup vs baseline: 8.8256x; 1.0698x over previous
import jax
import jax.numpy as jnp
from jax import lax
from jax.experimental import pallas as pl
from jax.experimental.pallas import tpu as pltpu
from jax.experimental.pallas import tpu_sc as plsc

_R = 3
_N = 10000
_NP = 10240
_E = 320000
_EP = 327680
_EROWS = _EP // 128
_TROWS = _EROWS // 16
_MR = 16
_MACROS = _TROWS // _MR
_ACC = _R * _NP
_STRIPE = _ACC // 16
_QW = 32


_NBUF = 6
_SLACK = 2
_GR = 32
_GROUPS = _TROWS // _GR


def _sc_body(xf, gidx, offs, sums, hists,
             offb, gixb, rows, hist,
             dsem0, dsem1, gsems, ssems, acc):
    c = lax.axis_index("c")
    s = lax.axis_index("s")
    zeros16 = jnp.zeros((16,), jnp.float32)
    ones16 = jnp.ones((16,), jnp.float32)

    def zh(r, carry):
        hist[pl.ds(r * 16, 16)] = zeros16
        return carry
    lax.fori_loop(0, 2048, zh, 0)

    base_a = s * _STRIPE

    for p in range(2):
        q = p * 2 + c

        def z0(r, carry):
            for w in range(2):
                rows[r, pl.ds(w * 16, 16)] = zeros16
            return carry
        lax.fori_loop(0, 128, z0, 0)

        zd = [pltpu.async_copy(rows.at[pl.ds(0, 128)],
                               acc.at[pl.ds(base_a + k * 128, 128)],
                               ssems.at[k % _NBUF])
              for k in range(_STRIPE // 128)]
        for d in zd:
            d.wait()

        plsc.subcore_barrier()

        def group(m, carry):
            base = s * _TROWS + m * _GR
            cp0 = pltpu.async_copy(offs.at[pl.ds(base, _GR)], offb, dsem0)
            cp1 = pltpu.async_copy(
                gidx.at[pl.ds(q * _EROWS + base, _GR)], gixb, dsem1)
            cp0.wait()
            cp1.wait()
            gd, sd = {}, {}
            for b in range(_NBUF):
                gd[b] = pltpu.async_copy(
                    xf.at[gixb.at[b]],
                    rows.at[pl.ds(b * 128, 128)], gsems.at[b])
            for j in range(_GR):
                b = j % _NBUF
                gd[j].wait()
                sd[j] = pltpu.async_copy(
                    rows.at[pl.ds(b * 128, 128)],
                    acc.at[offb.at[j]], ssems.at[b], add=True)
                if p == 0:
                    for w in range(8):
                        o = offb[j, pl.ds(w * 16, 16)]
                        plsc.addupdate_scatter(hist, [o], ones16)
                jj = j - _SLACK
                nxt = jj + _NBUF
                if jj >= 0 and nxt < _GR:
                    sd[jj].wait()
                    gd[nxt] = pltpu.async_copy(
                        xf.at[gixb.at[nxt]],
                        rows.at[pl.ds((jj % _NBUF) * 128, 128)],
                        gsems.at[jj % _NBUF])
            for j in range(_GR - _NBUF, _GR):
                sd[j].wait()
            return carry
        lax.fori_loop(0, _GROUPS, group, 0)

        plsc.subcore_barrier()

        pltpu.sync_copy(acc.at[pl.ds(base_a, _STRIPE)],
                        sums.at[pl.ds(q * _ACC + base_a, _STRIPE)])

    @pl.when(c == 0)
    def _():
        pltpu.sync_copy(hist, hists.at[s])


_sc_call = pl.kernel(
    _sc_body,
    out_type=[
        jax.ShapeDtypeStruct((4 * _ACC, _QW), jnp.float32),
        jax.ShapeDtypeStruct((16, 32768), jnp.float32),
    ],
    mesh=plsc.VectorSubcoreMesh(core_axis_name="c", subcore_axis_name="s"),
    compiler_params=pltpu.CompilerParams(
        needs_layout_passes=False, use_tc_tiling_on_sc=False),
    scratch_types=[
        pltpu.VMEM((_GR, 128), jnp.int32),
        pltpu.VMEM((_GR, 128), jnp.int32),
        pltpu.VMEM((_NBUF * 128, _QW), jnp.float32),
        pltpu.VMEM((32768,), jnp.float32),
        pltpu.SemaphoreType.DMA,
        pltpu.SemaphoreType.DMA,
        pltpu.SemaphoreType.DMA((_NBUF,)),
        pltpu.SemaphoreType.DMA((_NBUF,)),
        pltpu.VMEM_SHARED((_ACC, _QW), jnp.float32),
    ],
)


def _fin_body(x_ref, s_ref, cnt_ref, o_ref):
    cnt = jnp.sum(cnt_ref[...].reshape(16, _R, _FB), axis=0)
    x = x_ref[...]
    c0 = jnp.maximum(cnt[0], 1.0)[:, None]
    c1 = jnp.maximum(cnt[1], 1.0)[:, None]
    c2 = jnp.maximum(cnt[2], 1.0)[:, None]
    ctx = s_ref[0] / c0 + s_ref[1] / c1 + s_ref[2] / c2
    rc = (jnp.minimum(cnt[0], 1.0) + jnp.minimum(cnt[1], 1.0)
          + jnp.minimum(cnt[2], 1.0))
    rc = jnp.maximum(rc, 1.0)[:, None]
    o_ref[...] = x - ctx / rc


_FB = 1024

_fin_call = pl.pallas_call(
    _fin_body,
    grid=(_NP // _FB,),
    in_specs=[
        pl.BlockSpec((_FB, 128), lambda i: (i, 0)),
        pl.BlockSpec((_R, _FB, 128), lambda i: (0, i, 0)),
        pl.BlockSpec((48, _FB), lambda i: (0, i)),
    ],
    out_specs=pl.BlockSpec((_FB, 128), lambda i: (i, 0)),
    out_shape=jax.ShapeDtypeStruct((_NP, 128), jnp.float32),
)


def kernel(x, edge_index, edge_type):
    row = edge_index[0]
    col = edge_index[1]
    off = edge_type * _NP + row
    pad = _EP - _E
    dummy_off = _N + (jnp.arange(pad, dtype=jnp.int32) % 16)
    offp = jnp.concatenate([off, dummy_off]).reshape(_EROWS, 128)
    colp = jnp.concatenate([col, jnp.zeros((pad,), jnp.int32)])
    gidx = (colp[None, :]
            + (jnp.arange(4, dtype=jnp.int32) * _N)[:, None]
            ).reshape(4 * _EROWS, 128)
    xf = x.reshape(_N, 4, _QW).transpose(1, 0, 2).reshape(4 * _N, _QW)

    sums, hists = _sc_call(xf, gidx, offp)
    s = (sums.reshape(4, _R, _NP, _QW)
         .transpose(1, 2, 0, 3).reshape(_R, _NP, 128))
    cnt = hists[:, :_ACC].reshape(16 * _R, _NP)
    xpad = jnp.concatenate(
        [x, jnp.zeros((_NP - _N, 128), jnp.float32)], axis=0)
    out = _fin_call(xpad, s, cnt)
    return out[:_N]


# speedup vs baseline: 9.6504x; 1.0934x over previous
import jax
import jax.numpy as jnp
from jax import lax
from jax.experimental import pallas as pl
from jax.experimental.pallas import tpu as pltpu
from jax.experimental.pallas import tpu_sc as plsc

_R = 3
_N = 10000
_NP = 10240
_E = 320000
_EP = 327680
_EROWS = _EP // 128
_TROWS = _EROWS // 16
_MR = 16
_MACROS = _TROWS // _MR
_ACC = _R * _NP
_STRIPE = _ACC // 16
_QW = 32


_NBUF = 6
_SLACK = 2
_GR = 32
_GROUPS = _TROWS // _GR


def _sc_body(xf, gidx, offs, sums, hists,
             offb, gixb, rows, hist,
             dsem0, dsem1, gsems, ssems, acc):
    c = lax.axis_index("c")
    s = lax.axis_index("s")
    zeros16 = jnp.zeros((16,), jnp.float32)
    ones16 = jnp.ones((16,), jnp.float32)

    def zh(r, carry):
        hist[pl.ds(r * 16, 16)] = zeros16
        return carry
    lax.fori_loop(0, 2048, zh, 0)

    base_a = s * _STRIPE

    for p in range(2):
        q = p * 2 + c

        def z0(r, carry):
            for w in range(2):
                rows[r, pl.ds(w * 16, 16)] = zeros16
            return carry
        lax.fori_loop(0, 128, z0, 0)

        zd = [pltpu.async_copy(rows.at[pl.ds(0, 128)],
                               acc.at[pl.ds(base_a + k * 128, 128)],
                               ssems.at[k % _NBUF])
              for k in range(_STRIPE // 128)]
        for d in zd:
            d.wait()

        plsc.subcore_barrier()

        qn = q * _N

        def group(m, carry):
            base = s * _TROWS + m * _GR
            cp0 = pltpu.async_copy(offs.at[pl.ds(base, _GR)], offb, dsem0)
            cp1 = pltpu.async_copy(gidx.at[pl.ds(base, _GR)], gixb, dsem1)
            cp0.wait()
            cp1.wait()

            def addrow(r):
                for w in range(8):
                    gixb[r, pl.ds(w * 16, 16)] = (
                        gixb[r, pl.ds(w * 16, 16)] + qn)

            gd, sd = {}, {}
            for b in range(_NBUF):
                addrow(b)
                gd[b] = pltpu.async_copy(
                    xf.at[gixb.at[b]],
                    rows.at[pl.ds(b * 128, 128)], gsems.at[b])
            for j in range(_GR):
                b = j % _NBUF
                gd[j].wait()
                sd[j] = pltpu.async_copy(
                    rows.at[pl.ds(b * 128, 128)],
                    acc.at[offb.at[j]], ssems.at[b], add=True)
                if p == 0:
                    for w in range(8):
                        o = offb[j, pl.ds(w * 16, 16)]
                        plsc.addupdate_scatter(hist, [o], ones16)
                jj = j - _SLACK
                nxt = jj + _NBUF
                if jj >= 0 and nxt < _GR:
                    sd[jj].wait()
                    addrow(nxt)
                    gd[nxt] = pltpu.async_copy(
                        xf.at[gixb.at[nxt]],
                        rows.at[pl.ds((jj % _NBUF) * 128, 128)],
                        gsems.at[jj % _NBUF])
            for j in range(_GR - _NBUF, _GR):
                sd[j].wait()
            return carry
        lax.fori_loop(0, _GROUPS, group, 0)

        plsc.subcore_barrier()

        pltpu.sync_copy(acc.at[pl.ds(base_a, _STRIPE)],
                        sums.at[pl.ds(base_a, _STRIPE),
                                pl.ds(q * _QW, _QW)])

    @pl.when(c == 0)
    def _():
        pltpu.sync_copy(hist, hists.at[s])


_sc_call = pl.kernel(
    _sc_body,
    out_type=[
        jax.ShapeDtypeStruct((_ACC, 128), jnp.float32),
        jax.ShapeDtypeStruct((16, 32768), jnp.float32),
    ],
    mesh=plsc.VectorSubcoreMesh(core_axis_name="c", subcore_axis_name="s"),
    compiler_params=pltpu.CompilerParams(
        needs_layout_passes=False, use_tc_tiling_on_sc=False),
    scratch_types=[
        pltpu.VMEM((_GR, 128), jnp.int32),
        pltpu.VMEM((_GR, 128), jnp.int32),
        pltpu.VMEM((_NBUF * 128, _QW), jnp.float32),
        pltpu.VMEM((32768,), jnp.float32),
        pltpu.SemaphoreType.DMA,
        pltpu.SemaphoreType.DMA,
        pltpu.SemaphoreType.DMA((_NBUF,)),
        pltpu.SemaphoreType.DMA((_NBUF,)),
        pltpu.VMEM_SHARED((_ACC, _QW), jnp.float32),
    ],
)


def _fin_body(x_ref, s_ref, cnt_ref, o_ref):
    cnt48 = cnt_ref[...]
    x = x_ref[...]
    cnt = [cnt48[:, r:r + 1] for r in range(_R)]
    for t in range(1, 16):
        for r in range(_R):
            cnt[r] = cnt[r] + cnt48[:, 3 * t + r:3 * t + r + 1]
    c0 = jnp.maximum(cnt[0], 1.0)
    c1 = jnp.maximum(cnt[1], 1.0)
    c2 = jnp.maximum(cnt[2], 1.0)
    ctx = s_ref[0] / c0 + s_ref[1] / c1 + s_ref[2] / c2
    rc = (jnp.minimum(cnt[0], 1.0) + jnp.minimum(cnt[1], 1.0)
          + jnp.minimum(cnt[2], 1.0))
    rc = jnp.maximum(rc, 1.0)
    o_ref[...] = x - ctx / rc


_FB = 1000

_fin_call = pl.pallas_call(
    _fin_body,
    grid=(_N // _FB,),
    in_specs=[
        pl.BlockSpec((_FB, 128), lambda i: (i, 0)),
        pl.BlockSpec((_R, _FB, 128), lambda i: (0, i, 0)),
        pl.BlockSpec((_FB, 48), lambda i: (i, 0)),
    ],
    out_specs=pl.BlockSpec((_FB, 128), lambda i: (i, 0)),
    out_shape=jax.ShapeDtypeStruct((_N, 128), jnp.float32),
)


def kernel(x, edge_index, edge_type):
    row = edge_index[0]
    col = edge_index[1]
    off = edge_type * _NP + row
    pad = _EP - _E
    dummy_off = _N + (jnp.arange(pad, dtype=jnp.int32) % 16)
    offp = jnp.concatenate([off, dummy_off]).reshape(_EROWS, 128)
    colp = jnp.concatenate(
        [col, jnp.zeros((pad,), jnp.int32)]).reshape(_EROWS, 128)
    xf = x.reshape(_N, 4, _QW).transpose(1, 0, 2).reshape(4 * _N, _QW)

    sums, hists = _sc_call(xf, colp, offp)
    s = sums.reshape(_R, _NP, 128)
    cnt = (hists[:, :_ACC].reshape(16, _R, _NP)
           .transpose(2, 0, 1).reshape(_NP, 48))
    out = _fin_call(x, s, cnt)
    return out
